# Initial kernel scaffold; baseline (speedup 1.0000x reference)
import jax
import jax.numpy as jnp
from jax.experimental import pallas as pl


def kernel(x, Wg, W1, W2, Wp, W1s, W2s, Wps, Wsg):
    raise NotImplementedError("write your pallas kernel here")


# speedup vs baseline: 1.6430x; 1.6430x over previous
import functools

import jax
import jax.numpy as jnp
from jax import lax
from jax.experimental import pallas as pl
from jax.experimental.pallas import tpu as pltpu
from jax.experimental.pallas import tpu_sc as plsc

E = 16
TOPK = 2
C = 1024
I = 704
IS = 2816
T = 2048
BLK = 128
NB = (TOPK * T + E * (BLK - 1) + BLK - 1) // BLK
R_MAX = NB * BLK
IS_B = 256
JS = IS // IS_B
TB = 256
PAIRS = TOPK * T


def _router_body(xf_ref, wg_ref, p1_ref, p2_ref, d1_ref, d2_ref,
                 be_ref, ba_ref):
    xf = xf_ref[...]
    wg = wg_ref[...]
    logits = lax.dot_general(xf, wg, (((1,), (1,)), ((), ())),
                             preferred_element_type=jnp.float32)
    m = jnp.max(logits, axis=1, keepdims=True)
    ex = jnp.exp(logits - m)
    p = ex / jnp.sum(ex, axis=1, keepdims=True)
    lane = lax.broadcasted_iota(jnp.int32, (T, E), 1)
    m1 = jnp.max(p, axis=1, keepdims=True)
    a1 = jnp.min(jnp.where(p == m1, lane, E), axis=1, keepdims=True)
    pm = jnp.where(lane == a1, -1.0, p)
    m2 = jnp.max(pm, axis=1, keepdims=True)
    a2 = jnp.min(jnp.where(pm == m2, lane, E), axis=1, keepdims=True)

    oh1 = (lane == a1).astype(jnp.float32)
    oh2 = (lane == a2).astype(jnp.float32)
    ri = lax.broadcasted_iota(jnp.int32, (T, T), 0)
    ci = lax.broadcasted_iota(jnp.int32, (T, T), 1)
    slt = (ci < ri).astype(jnp.bfloat16)
    P1 = lax.dot_general(slt, oh1.astype(jnp.bfloat16),
                         (((1,), (0,)), ((), ())),
                         preferred_element_type=jnp.float32)
    P2 = lax.dot_general(slt, oh2.astype(jnp.bfloat16),
                         (((1,), (0,)), ((), ())),
                         preferred_element_type=jnp.float32)
    c1 = jnp.sum(oh1, axis=0, keepdims=True)
    c2 = jnp.sum(oh2, axis=0, keepdims=True)
    n = c1 + c2
    blocks = jnp.floor((n + (BLK - 1)) * (1.0 / BLK))
    ei = lax.broadcasted_iota(jnp.int32, (E, E), 0)
    ej = lax.broadcasted_iota(jnp.int32, (E, E), 1)
    ile = (ei <= ej).astype(jnp.float32)
    cum = lax.dot_general(blocks, ile, (((1,), (0,)), ((), ())),
                          preferred_element_type=jnp.float32)
    boff = BLK * (cum - blocks)
    rank1 = jnp.sum(oh1 * P1, axis=1, keepdims=True)
    rank2 = jnp.sum(oh2 * (P2 + c1), axis=1, keepdims=True)
    d1 = jnp.sum(oh1 * boff, axis=1, keepdims=True) + rank1
    d2 = jnp.sum(oh2 * boff, axis=1, keepdims=True) + rank2

    total = jnp.max(cum, axis=1, keepdims=True)
    bid = lax.broadcasted_iota(jnp.int32, (NB, E), 0).astype(jnp.float32)
    bidc = jnp.minimum(bid, total - 1.0)
    be = jnp.sum((bidc >= cum).astype(jnp.int32), axis=1, keepdims=True)
    ba = (lax.broadcasted_iota(jnp.int32, (NB, 1), 0).astype(jnp.float32)
          < total).astype(jnp.int32)

    p1_ref[...] = m1
    p2_ref[...] = m2
    d1_ref[...] = d1.astype(jnp.int32)
    d2_ref[...] = d2.astype(jnp.int32)
    be_ref[...] = be
    ba_ref[...] = ba


def _router(xf, wg):
    return pl.pallas_call(
        _router_body,
        out_shape=(
            jax.ShapeDtypeStruct((T, 1), jnp.float32),
            jax.ShapeDtypeStruct((T, 1), jnp.float32),
            jax.ShapeDtypeStruct((T, 1), jnp.int32),
            jax.ShapeDtypeStruct((T, 1), jnp.int32),
            jax.ShapeDtypeStruct((NB, 1), jnp.int32),
            jax.ShapeDtypeStruct((NB, 1), jnp.int32),
        ),
    )(xf, wg)


def _group_body(be_ref, ba_ref, xs_ref, w1_ref, w2_ref, wp_ref, out_ref):
    @pl.when(ba_ref[pl.program_id(0)] == 1)
    def _():
        xb = xs_ref[...]
        h1 = lax.dot_general(xb, w1_ref[0], (((1,), (1,)), ((), ())),
                             preferred_element_type=jnp.float32)
        h2 = lax.dot_general(xb, w2_ref[0], (((1,), (1,)), ((), ())),
                             preferred_element_type=jnp.float32)
        h = (h1 * jax.nn.sigmoid(h1)) * h2
        out_ref[...] = lax.dot_general(h, wp_ref[0], (((1,), (1,)), ((), ())),
                                       preferred_element_type=jnp.float32)


def _grouped(be, ba, xs, w1, w2, wp):
    grid_spec = pltpu.PrefetchScalarGridSpec(
        num_scalar_prefetch=2,
        grid=(NB,),
        in_specs=[
            pl.BlockSpec((BLK, C), lambda b, be, ba: (b, 0)),
            pl.BlockSpec((1, I, C), lambda b, be, ba: (be[b], 0, 0)),
            pl.BlockSpec((1, I, C), lambda b, be, ba: (be[b], 0, 0)),
            pl.BlockSpec((1, C, I), lambda b, be, ba: (be[b], 0, 0)),
        ],
        out_specs=pl.BlockSpec((BLK, C), lambda b, be, ba: (b, 0)),
    )
    return pl.pallas_call(
        _group_body,
        grid_spec=grid_spec,
        out_shape=jax.ShapeDtypeStruct((R_MAX, C), jnp.float32),
    )(be, ba, xs, w1, w2, wp)


def _shared_body(xf_ref, w1s_ref, w2s_ref, wps_ref, wsg_ref,
                 y1_ref, y2_ref, p1_ref, p2_ref, out_ref, acc_ref):
    j = pl.program_id(1)
    xb = xf_ref[...]
    h1 = lax.dot_general(xb, w1s_ref[...], (((1,), (1,)), ((), ())),
                         preferred_element_type=jnp.float32)
    h2 = lax.dot_general(xb, w2s_ref[...], (((1,), (1,)), ((), ())),
                         preferred_element_type=jnp.float32)
    h = (h1 * jax.nn.sigmoid(h1)) * h2
    part = lax.dot_general(h, wps_ref[...], (((1,), (1,)), ((), ())),
                           preferred_element_type=jnp.float32)

    @pl.when(j == 0)
    def _():
        acc_ref[...] = part

    @pl.when(j > 0)
    def _():
        acc_ref[...] = acc_ref[...] + part

    @pl.when(j == JS - 1)
    def _():
        g = jax.nn.sigmoid(
            lax.dot_general(xb, wsg_ref[...], (((1,), (1,)), ((), ())),
                            preferred_element_type=jnp.float32))
        out_ref[...] = (g * acc_ref[...] + p1_ref[...] * y1_ref[...]
                        + p2_ref[...] * y2_ref[...])


def _shared_combine(xf, w1s, w2s, wps, wsg, y1, y2, p1, p2):
    return pl.pallas_call(
        _shared_body,
        grid=(T // TB, JS),
        in_specs=[
            pl.BlockSpec((TB, C), lambda t, j: (t, 0)),
            pl.BlockSpec((IS_B, C), lambda t, j: (j, 0)),
            pl.BlockSpec((IS_B, C), lambda t, j: (j, 0)),
            pl.BlockSpec((C, IS_B), lambda t, j: (0, j)),
            pl.BlockSpec((1, C), lambda t, j: (0, 0)),
            pl.BlockSpec((TB, C), lambda t, j: (t, 0)),
            pl.BlockSpec((TB, C), lambda t, j: (t, 0)),
            pl.BlockSpec((TB, 1), lambda t, j: (t, 0)),
            pl.BlockSpec((TB, 1), lambda t, j: (t, 0)),
        ],
        out_specs=pl.BlockSpec((TB, C), lambda t, j: (t, 0)),
        out_shape=jax.ShapeDtypeStruct((T, C), jnp.float32),
        scratch_shapes=[pltpu.VMEM((TB, C), jnp.float32)],
    )(xf, w1s, w2s, wps, wsg, y1, y2, p1, p2)


NW = 32
PPW = PAIRS // NW
SCH = 32
SNCH = PPW // SCH


def _sc_scatter(xf_hbm, dest_hbm, xs_hbm, idx_v, rows_v, sem):
    wid = lax.axis_index("s") * 2 + lax.axis_index("c")

    def body(c, _):
        base = wid * PPW + c * SCH
        row = jnp.where(base >= T, base - T, base)
        pltpu.sync_copy(dest_hbm.at[pl.ds(base, SCH)], idx_v)
        pltpu.sync_copy(xf_hbm.at[pl.ds(row, SCH)], rows_v)
        pltpu.async_copy(rows_v, xs_hbm.at[idx_v], sem).wait()
        return 0

    lax.fori_loop(0, SNCH, body, 0)


def _scatter_rows(xf, dest):
    mesh = plsc.VectorSubcoreMesh(core_axis_name="c", subcore_axis_name="s")
    k = functools.partial(
        pl.kernel, mesh=mesh,
        out_type=jax.ShapeDtypeStruct((R_MAX, C), jnp.float32),
        scratch_types=[
            pltpu.VMEM((SCH,), jnp.int32),
            pltpu.VMEM((SCH, C), jnp.float32),
            pltpu.SemaphoreType.DMA,
        ],
    )(_sc_scatter)
    return k(xf, dest)


TPW = T // NW
GCH = 16
GNCH = TPW // GCH


def _sc_gather(orows_hbm, d1_hbm, d2_hbm, y1_hbm, y2_hbm, idx_v, buf_v, sem):
    wid = lax.axis_index("s") * 2 + lax.axis_index("c")

    def body(c, _):
        tok = wid * TPW + c * GCH
        pltpu.sync_copy(d1_hbm.at[pl.ds(tok, GCH)], idx_v)
        pltpu.async_copy(orows_hbm.at[idx_v], buf_v, sem).wait()
        pltpu.sync_copy(buf_v, y1_hbm.at[pl.ds(tok, GCH)])
        pltpu.sync_copy(d2_hbm.at[pl.ds(tok, GCH)], idx_v)
        pltpu.async_copy(orows_hbm.at[idx_v], buf_v, sem).wait()
        pltpu.sync_copy(buf_v, y2_hbm.at[pl.ds(tok, GCH)])
        return 0

    lax.fori_loop(0, GNCH, body, 0)


def _gather_rows(orows, d1, d2):
    mesh = plsc.VectorSubcoreMesh(core_axis_name="c", subcore_axis_name="s")
    k = functools.partial(
        pl.kernel, mesh=mesh,
        out_type=(
            jax.ShapeDtypeStruct((T, C), jnp.float32),
            jax.ShapeDtypeStruct((T, C), jnp.float32),
        ),
        scratch_types=[
            pltpu.VMEM((GCH,), jnp.int32),
            pltpu.VMEM((GCH, C), jnp.float32),
            pltpu.SemaphoreType.DMA,
        ],
    )(_sc_gather)
    return k(orows, d1, d2)


def kernel(x, Wg, W1, W2, Wp, W1s, W2s, Wps, Wsg):
    B, Tt, Cc = x.shape
    xf = x.reshape(T, C)
    p1, p2, d1, d2, be, ba = _router(xf, Wg)
    dest = jnp.concatenate([d1.reshape(T), d2.reshape(T)], axis=0)
    xs = _scatter_rows(xf, dest)
    orows = _grouped(be.reshape(NB), ba.reshape(NB), xs, W1, W2, Wp)
    y1, y2 = _gather_rows(orows, d1.reshape(T), d2.reshape(T))
    y = _shared_combine(xf, W1s, W2s, Wps, Wsg, y1, y2, p1, p2)
    return y.reshape(B, Tt, Cc)


# speedup vs baseline: 2.1500x; 1.3086x over previous
import functools

import jax
import jax.numpy as jnp
from jax import lax
from jax.experimental import pallas as pl
from jax.experimental.pallas import tpu as pltpu
from jax.experimental.pallas import tpu_sc as plsc

E = 16
TOPK = 2
C = 1024
I = 704
IS = 2816
T = 2048
BLK = 128
NB = (TOPK * T + E * (BLK - 1) + BLK - 1) // BLK
R_MAX = NB * BLK
IS_B = 256
JS = IS // IS_B
TB = 256
PAIRS = TOPK * T


def _router_body(xf_ref, wg_ref, p1_ref, p2_ref, d1_ref, d2_ref,
                 be_ref, ba_ref):
    xf = xf_ref[...]
    wg = wg_ref[...]
    logits = lax.dot_general(xf, wg, (((1,), (1,)), ((), ())),
                             preferred_element_type=jnp.float32)
    m = jnp.max(logits, axis=1, keepdims=True)
    ex = jnp.exp(logits - m)
    p = ex / jnp.sum(ex, axis=1, keepdims=True)
    lane = lax.broadcasted_iota(jnp.int32, (T, E), 1)
    m1 = jnp.max(p, axis=1, keepdims=True)
    a1 = jnp.min(jnp.where(p == m1, lane, E), axis=1, keepdims=True)
    pm = jnp.where(lane == a1, -1.0, p)
    m2 = jnp.max(pm, axis=1, keepdims=True)
    a2 = jnp.min(jnp.where(pm == m2, lane, E), axis=1, keepdims=True)

    oh1 = (lane == a1).astype(jnp.float32)
    oh2 = (lane == a2).astype(jnp.float32)
    ri = lax.broadcasted_iota(jnp.int32, (T, T), 0)
    ci = lax.broadcasted_iota(jnp.int32, (T, T), 1)
    slt = (ci < ri).astype(jnp.bfloat16)
    P1 = lax.dot_general(slt, oh1.astype(jnp.bfloat16),
                         (((1,), (0,)), ((), ())),
                         preferred_element_type=jnp.float32)
    P2 = lax.dot_general(slt, oh2.astype(jnp.bfloat16),
                         (((1,), (0,)), ((), ())),
                         preferred_element_type=jnp.float32)
    c1 = jnp.sum(oh1, axis=0, keepdims=True)
    c2 = jnp.sum(oh2, axis=0, keepdims=True)
    n = c1 + c2
    blocks = jnp.floor((n + (BLK - 1)) * (1.0 / BLK))
    ei = lax.broadcasted_iota(jnp.int32, (E, E), 0)
    ej = lax.broadcasted_iota(jnp.int32, (E, E), 1)
    ile = (ei <= ej).astype(jnp.float32)
    cum = lax.dot_general(blocks, ile, (((1,), (0,)), ((), ())),
                          preferred_element_type=jnp.float32)
    boff = BLK * (cum - blocks)
    rank1 = jnp.sum(oh1 * P1, axis=1, keepdims=True)
    rank2 = jnp.sum(oh2 * (P2 + c1), axis=1, keepdims=True)
    d1 = jnp.sum(oh1 * boff, axis=1, keepdims=True) + rank1
    d2 = jnp.sum(oh2 * boff, axis=1, keepdims=True) + rank2

    total = jnp.max(cum, axis=1, keepdims=True)
    bid = lax.broadcasted_iota(jnp.int32, (NB, E), 0).astype(jnp.float32)
    bidc = jnp.minimum(bid, total - 1.0)
    be = jnp.sum((bidc >= cum).astype(jnp.int32), axis=1, keepdims=True)
    ba = (lax.broadcasted_iota(jnp.int32, (NB, 1), 0).astype(jnp.float32)
          < total).astype(jnp.int32)

    p1_ref[...] = m1
    p2_ref[...] = m2
    d1_ref[...] = d1.astype(jnp.int32)
    d2_ref[...] = d2.astype(jnp.int32)
    be_ref[...] = be
    ba_ref[...] = ba


def _router(xf, wg):
    return pl.pallas_call(
        _router_body,
        out_shape=(
            jax.ShapeDtypeStruct((T, 1), jnp.float32),
            jax.ShapeDtypeStruct((T, 1), jnp.float32),
            jax.ShapeDtypeStruct((T, 1), jnp.int32),
            jax.ShapeDtypeStruct((T, 1), jnp.int32),
            jax.ShapeDtypeStruct((NB, 1), jnp.int32),
            jax.ShapeDtypeStruct((NB, 1), jnp.int32),
        ),
    )(xf, wg)


def _group_body(be_ref, ba_ref, xs_ref, w1_ref, w2_ref, wp_ref, out_ref):
    @pl.when(ba_ref[pl.program_id(0)] == 1)
    def _():
        xb = xs_ref[...].astype(jnp.bfloat16)
        h1 = lax.dot_general(xb, w1_ref[0].astype(jnp.bfloat16),
                             (((1,), (1,)), ((), ())),
                             preferred_element_type=jnp.float32)
        h2 = lax.dot_general(xb, w2_ref[0].astype(jnp.bfloat16),
                             (((1,), (1,)), ((), ())),
                             preferred_element_type=jnp.float32)
        h = ((h1 * jax.nn.sigmoid(h1)) * h2).astype(jnp.bfloat16)
        out_ref[...] = lax.dot_general(h, wp_ref[0].astype(jnp.bfloat16),
                                       (((1,), (1,)), ((), ())),
                                       preferred_element_type=jnp.float32)


def _grouped(be, ba, xs, w1, w2, wp):
    grid_spec = pltpu.PrefetchScalarGridSpec(
        num_scalar_prefetch=2,
        grid=(NB,),
        in_specs=[
            pl.BlockSpec((BLK, C), lambda b, be, ba: (b * ba[b], 0)),
            pl.BlockSpec((1, I, C), lambda b, be, ba: (be[b], 0, 0)),
            pl.BlockSpec((1, I, C), lambda b, be, ba: (be[b], 0, 0)),
            pl.BlockSpec((1, C, I), lambda b, be, ba: (be[b], 0, 0)),
        ],
        out_specs=pl.BlockSpec((BLK, C), lambda b, be, ba: (b, 0)),
    )
    return pl.pallas_call(
        _group_body,
        grid_spec=grid_spec,
        out_shape=jax.ShapeDtypeStruct((R_MAX, C), jnp.float32),
    )(be, ba, xs, w1, w2, wp)


def _shared_body(xf_ref, w1s_ref, w2s_ref, wps_ref, wsg_ref,
                 y1_ref, y2_ref, p1_ref, p2_ref, out_ref):
    xb = xf_ref[...]
    xbb = xb.astype(jnp.bfloat16)
    h1 = lax.dot_general(xbb, w1s_ref[...].astype(jnp.bfloat16),
                         (((1,), (1,)), ((), ())),
                         preferred_element_type=jnp.float32)
    h2 = lax.dot_general(xbb, w2s_ref[...].astype(jnp.bfloat16),
                         (((1,), (1,)), ((), ())),
                         preferred_element_type=jnp.float32)
    h = ((h1 * jax.nn.sigmoid(h1)) * h2).astype(jnp.bfloat16)
    sh = lax.dot_general(h, wps_ref[...].astype(jnp.bfloat16),
                         (((1,), (1,)), ((), ())),
                         preferred_element_type=jnp.float32)
    g = jax.nn.sigmoid(
        lax.dot_general(xb, wsg_ref[...], (((1,), (1,)), ((), ())),
                        preferred_element_type=jnp.float32))
    out_ref[...] = (g * sh + p1_ref[...] * y1_ref[...]
                    + p2_ref[...] * y2_ref[...])


def _shared_combine(xf, w1s, w2s, wps, wsg, y1, y2, p1, p2):
    return pl.pallas_call(
        _shared_body,
        grid=(T // TB,),
        in_specs=[
            pl.BlockSpec((TB, C), lambda t: (t, 0)),
            pl.BlockSpec((IS, C), lambda t: (0, 0)),
            pl.BlockSpec((IS, C), lambda t: (0, 0)),
            pl.BlockSpec((C, IS), lambda t: (0, 0)),
            pl.BlockSpec((1, C), lambda t: (0, 0)),
            pl.BlockSpec((TB, C), lambda t: (t, 0)),
            pl.BlockSpec((TB, C), lambda t: (t, 0)),
            pl.BlockSpec((TB, 1), lambda t: (t, 0)),
            pl.BlockSpec((TB, 1), lambda t: (t, 0)),
        ],
        out_specs=pl.BlockSpec((TB, C), lambda t: (t, 0)),
        out_shape=jax.ShapeDtypeStruct((T, C), jnp.float32),
    )(xf, w1s, w2s, wps, wsg, y1, y2, p1, p2)


NW = 32
PPW = PAIRS // NW
SCH = 32
SNCH = PPW // SCH


def _sc_scatter(xf_hbm, dest_hbm, xs_hbm, idx_v, rows_v, sem):
    wid = lax.axis_index("s") * 2 + lax.axis_index("c")

    def body(c, _):
        base = wid * PPW + c * SCH
        row = jnp.where(base >= T, base - T, base)
        pltpu.sync_copy(dest_hbm.at[pl.ds(base, SCH)], idx_v)
        pltpu.sync_copy(xf_hbm.at[pl.ds(row, SCH)], rows_v)
        pltpu.async_copy(rows_v, xs_hbm.at[idx_v], sem).wait()
        return 0

    lax.fori_loop(0, SNCH, body, 0)


def _scatter_rows(xf, dest):
    mesh = plsc.VectorSubcoreMesh(core_axis_name="c", subcore_axis_name="s")
    k = functools.partial(
        pl.kernel, mesh=mesh,
        out_type=jax.ShapeDtypeStruct((R_MAX, C), jnp.float32),
        scratch_types=[
            pltpu.VMEM((SCH,), jnp.int32),
            pltpu.VMEM((SCH, C), jnp.float32),
            pltpu.SemaphoreType.DMA,
        ],
    )(_sc_scatter)
    return k(xf, dest)


TPW = T // NW
GCH = 16
GNCH = TPW // GCH


def _sc_gather(orows_hbm, d1_hbm, d2_hbm, y1_hbm, y2_hbm, idx_v, buf_v, sem):
    wid = lax.axis_index("s") * 2 + lax.axis_index("c")

    def body(c, _):
        tok = wid * TPW + c * GCH
        pltpu.sync_copy(d1_hbm.at[pl.ds(tok, GCH)], idx_v)
        pltpu.async_copy(orows_hbm.at[idx_v], buf_v, sem).wait()
        pltpu.sync_copy(buf_v, y1_hbm.at[pl.ds(tok, GCH)])
        pltpu.sync_copy(d2_hbm.at[pl.ds(tok, GCH)], idx_v)
        pltpu.async_copy(orows_hbm.at[idx_v], buf_v, sem).wait()
        pltpu.sync_copy(buf_v, y2_hbm.at[pl.ds(tok, GCH)])
        return 0

    lax.fori_loop(0, GNCH, body, 0)


def _gather_rows(orows, d1, d2):
    mesh = plsc.VectorSubcoreMesh(core_axis_name="c", subcore_axis_name="s")
    k = functools.partial(
        pl.kernel, mesh=mesh,
        out_type=(
            jax.ShapeDtypeStruct((T, C), jnp.float32),
            jax.ShapeDtypeStruct((T, C), jnp.float32),
        ),
        scratch_types=[
            pltpu.VMEM((GCH,), jnp.int32),
            pltpu.VMEM((GCH, C), jnp.float32),
            pltpu.SemaphoreType.DMA,
        ],
    )(_sc_gather)
    return k(orows, d1, d2)


def kernel(x, Wg, W1, W2, Wp, W1s, W2s, Wps, Wsg):
    B, Tt, Cc = x.shape
    xf = x.reshape(T, C)
    p1, p2, d1, d2, be, ba = _router(xf, Wg)
    dest = jnp.concatenate([d1.reshape(T), d2.reshape(T)], axis=0)
    xs = _scatter_rows(xf, dest)
    orows = _grouped(be.reshape(NB), ba.reshape(NB), xs, W1, W2, Wp)
    y1, y2 = _gather_rows(orows, d1.reshape(T), d2.reshape(T))
    y = _shared_combine(xf, W1s, W2s, Wps, Wsg, y1, y2, p1, p2)
    return y.reshape(B, Tt, Cc)
